# baseline (speedup 1.0000x reference)
import jax
import jax.numpy as jnp
from jax.experimental import pallas as pl
from jax.experimental.pallas import tpu as pltpu

EPS = 1e-06
NUM_GROUPS = 4
BLOCK_T = 2048


def _glnorm_kernel(x_ref, tt_ref, gb_ref, o_ref):
    x = x_ref[...]
    tt = tt_ref[...]
    d = x.shape[1]
    mean = jnp.mean(x, axis=1, keepdims=True)
    xc = x - mean
    var = jnp.mean(xc * xc, axis=1, keepdims=True)
    inv = jax.lax.rsqrt(var + EPS)
    onehot = (tt == jnp.arange(NUM_GROUPS)[None, :]).astype(jnp.float32)
    gb = jax.lax.dot(onehot, gb_ref[...])
    o_ref[...] = xc * (inv * gb[:, :d]) + gb[:, d:]


def kernel(x, token_types, gamma, beta):
    B, S, D = x.shape
    n_tok = B * S
    x2 = x.reshape(n_tok, D)
    tt2 = token_types.reshape(n_tok, 1).astype(jnp.int32)
    gb = jnp.concatenate([gamma, beta], axis=1)
    grid = (n_tok // BLOCK_T,)
    out = pl.pallas_call(
        _glnorm_kernel,
        grid=grid,
        in_specs=[
            pl.BlockSpec((BLOCK_T, D), lambda i: (i, 0)),
            pl.BlockSpec((BLOCK_T, 1), lambda i: (i, 0)),
            pl.BlockSpec((NUM_GROUPS, 2 * D), lambda i: (0, 0)),
        ],
        out_specs=pl.BlockSpec((BLOCK_T, D), lambda i: (i, 0)),
        out_shape=jax.ShapeDtypeStruct((n_tok, D), x.dtype),
        compiler_params=pltpu.CompilerParams(vmem_limit_bytes=110 * 1024 * 1024),
    )(x2, tt2, gb)
    return out.reshape(B, S, D)
